# baseline (speedup 1.0000x reference)
import functools

import jax
import jax.numpy as jnp
from jax import lax
from jax.experimental import pallas as pl
from jax.experimental.pallas import tpu as pltpu
from jax.experimental.pallas import tpu_sc as plsc

NSEG = 500
NC = 2
NS = 16
NW = NC * NS
LANES = 16
SEG_PT = 16

N_TOTAL = 6400000
CHUNK = 20000
NSAMP = 1280
SSTRIDE = N_TOTAL // NSAMP
WWIN = 336
UNROLL = 8


def _make_sc_kernel():
  mesh = plsc.VectorSubcoreMesh(core_axis_name="c", subcore_axis_name="s")

  @functools.partial(
      pl.kernel,
      out_type=jax.ShapeDtypeStruct((NW, SEG_PT), jnp.float32),
      mesh=mesh,
      compiler_params=pltpu.CompilerParams(needs_layout_passes=False),
      scratch_types=[
          pltpu.VMEM((CHUNK,), jnp.float32),
          pltpu.VMEM((CHUNK,), jnp.float32),
          pltpu.VMEM((NSAMP,), jnp.int32),
          pltpu.VMEM((NSAMP,), jnp.int32),
          pltpu.VMEM((256,), jnp.int32),
          pltpu.VMEM((256,), jnp.int32),
          pltpu.VMEM((17 * WWIN,), jnp.int32),
          pltpu.VMEM((SEG_PT * LANES,), jnp.float32),
          pltpu.VMEM((SEG_PT,), jnp.float32),
          pltpu.SemaphoreType.DMA,
          pltpu.SemaphoreType.DMA,
          pltpu.SemaphoreType.DMA,
      ],
  )
  def seg_sum(in_hbm, lab_hbm, out_hbm, in0, in1, idxw, sbuf, pidx, pg,
              wbuf, acc, part, sem0, sem1, gsem):
    cid = lax.axis_index("c")
    sid = lax.axis_index("s")
    wid = sid * NC + cid
    lane = lax.iota(jnp.int32, LANES)
    zf = jnp.zeros((LANES,), jnp.float32)

    for s in range(SEG_PT):
      acc[pl.ds(s * LANES, LANES)] = zf

    c_a = wid * SEG_PT + lane
    c_b = jnp.full((LANES,), wid * SEG_PT + SEG_PT, jnp.int32)

    def widx_body(i, carry):
      idxw[pl.ds(i * LANES, LANES)] = (i * LANES + lane) * SSTRIDE
      return carry

    lax.fori_loop(0, NSAMP // LANES, widx_body, 0)
    whs = []
    for j in range(NSAMP // 128):
      whs.append(pltpu.async_copy(
          lab_hbm.at[idxw.at[pl.ds(j * 128, 128)]],
          sbuf.at[pl.ds(j * 128, 128)], gsem))
    for h in whs:
      h.wait()

    def sbin_body(r, carry):
      lo_a, hi_a, lo_b, hi_b = carry

      def step(lo, hi, c):
        mid = (lo + hi) // 2
        sv = plsc.load_gather(sbuf, [jnp.minimum(mid, NSAMP - 1)])
        sel = sv < c
        return jnp.where(sel, mid + 1, lo), jnp.where(sel, hi, mid)

      lo_a, hi_a = step(lo_a, hi_a, c_a)
      lo_b, hi_b = step(lo_b, hi_b, c_b)
      return lo_a, hi_a, lo_b, hi_b

    zi = jnp.zeros((LANES,), jnp.int32)
    mi = jnp.full((LANES,), NSAMP, jnp.int32)
    k_a, _, k_b, _ = lax.fori_loop(0, 11, sbin_body, (zi, mi, zi, mi))
    blo_a = jnp.maximum(k_a - 1, 0) * SSTRIDE
    bhi_a = jnp.minimum(k_a * SSTRIDE, N_TOTAL)
    blo_b = jnp.maximum(k_b - 1, 0) * SSTRIDE
    bhi_b = jnp.minimum(k_b * SSTRIDE, N_TOTAL)

    w_a = bhi_a - blo_a
    w_b = bhi_b - blo_b
    for k in range(1, 16):
      p_a = blo_a + (w_a * k) // 16
      pidx[pl.ds((k - 1) * LANES, LANES)] = jnp.minimum(p_a, N_TOTAL - 1)
    p_b = blo_b + (w_b * (lane + 1)) // 16
    pidx[pl.ds(240, LANES)] = jnp.minimum(p_b, N_TOTAL - 1)
    h1 = pltpu.async_copy(lab_hbm.at[pidx.at[pl.ds(0, 128)]],
                          pg.at[pl.ds(0, 128)], gsem)
    h2 = pltpu.async_copy(lab_hbm.at[pidx.at[pl.ds(128, 128)]],
                          pg.at[pl.ds(128, 128)], gsem)
    h1.wait()
    h2.wait()
    cnt_a = jnp.zeros((LANES,), jnp.int32)
    for k in range(1, 16):
      cnt_a += jnp.where(pg[pl.ds((k - 1) * LANES, LANES)] < c_a, 1, 0)
    selb = (pg[pl.ds(240, LANES)] < c_b) & (lane < 15)
    cnt_b = plsc.all_reduce_population_count(selb)

    def upd16(lo, hi, w, cnt):
      nlo = jnp.where(cnt == 0, lo, lo + (w * cnt) // 16 + 1)
      nhi = jnp.where(cnt == 15, hi, lo + (w * (cnt + 1)) // 16)
      return jnp.where(w > 0, nlo, lo), jnp.where(w > 0, nhi, hi)

    blo_a, bhi_a = upd16(blo_a, bhi_a, w_a, cnt_a)
    blo_b, bhi_b = upd16(blo_b, bhi_b, w_b, cnt_b)

    flo_a = jnp.minimum((blo_a // 8) * 8, N_TOTAL - WWIN)
    flo_b = jnp.minimum((blo_b // 8) * 8, N_TOTAL - WWIN)
    whs = []
    for t in range(16):
      whs.append(pltpu.async_copy(
          lab_hbm.at[pl.ds(pl.multiple_of(flo_a[t], 8), WWIN)],
          wbuf.at[pl.ds(t * WWIN, WWIN)], gsem))
    whs.append(pltpu.async_copy(
        lab_hbm.at[pl.ds(pl.multiple_of(flo_b[0], 8), WWIN)],
        wbuf.at[pl.ds(16 * WWIN, WWIN)], gsem))
    for h in whs:
      h.wait()

    def wcount(t, c_scalar):
      cv = jnp.full((LANES,), c_scalar, jnp.int32)

      def cbody(i, cnt):
        sv = wbuf[pl.ds(t * WWIN + i * LANES, LANES)]
        return cnt + jnp.where(sv < cv, 1, 0)

      cnt = lax.fori_loop(0, WWIN // LANES, cbody, zi)
      return jnp.sum(cnt)

    b_list = []
    for t in range(16):
      b_list.append(flo_a[t] + wcount(t, wid * SEG_PT + t))
    b_end = flo_b[0] + wcount(16, wid * SEG_PT + SEG_PT)

    bn = jnp.where(lane == 15, b_end, 0)
    for t in range(1, 16):
      bn = bn + jnp.where(lane == t - 1, b_list[t], 0)
    estart = b_list[0]
    eend = b_end

    astart = (estart // LANES) * LANES
    aend = jnp.minimum(((eend + LANES - 1) // LANES) * LANES, N_TOTAL)
    nch = (aend - astart + CHUNK - 1) // CHUNK

    bufs = ((in0, sem0), (in1, sem1))

    def cstart(k, sl):
      buf, sem = bufs[sl]
      off = jnp.minimum(astart + k * CHUNK, N_TOTAL - CHUNK)
      pltpu.async_copy(in_hbm.at[pl.ds(off, CHUNK)], buf, sem)

    def cdrain(sl):
      buf, sem = bufs[sl]
      pltpu.make_async_copy(in_hbm.at[pl.ds(0, CHUNK)], buf, sem).wait()

    @pl.when(nch > 0)
    def _():
      cstart(0, 0)

    @pl.when(nch > 1)
    def _():
      cstart(1, 1)

    def process(k, sl):
      buf, _ = bufs[sl]
      plo = astart + k * CHUNK
      chunk_lo = jnp.minimum(plo, N_TOTAL - CHUNK)
      pend = jnp.minimum(plo + CHUNK, eend)
      pos0 = jnp.maximum(estart, plo)

      def run_cond(pos):
        return pos < pend

      def run_body(pos):
        s = plsc.all_reduce_population_count(bn <= pos)[0]
        rend = jnp.minimum(jnp.min(jnp.where(bn > pos, bn, N_TOTAL)), pend)
        arow = acc.at[pl.ds(s * LANES, LANES)]
        q0 = pos - chunk_lo
        q1 = rend - chunk_lo
        hbase = (q0 // LANES) * LANES
        hpos = hbase + lane
        hm = (hpos >= q0) & (hpos < q1)
        plsc.addupdate(arow, jnp.where(hm, buf[pl.ds(hbase, LANES)], 0.0))
        m0 = hbase + LANES
        a1 = (q1 // LANES) * LANES

        @pl.when(m0 < a1)
        def _():
          @plsc.parallel_loop(m0 // LANES, a1 // LANES, unroll=UNROLL)
          def _(i):
            plsc.addupdate(arow, buf[pl.ds(i * LANES, LANES)])

        @pl.when((a1 < q1) & (a1 >= m0))
        def _():
          tm = (a1 + lane) < q1
          plsc.addupdate(arow, jnp.where(tm, buf[pl.ds(a1, LANES)], 0.0))

        return rend

      lax.while_loop(run_cond, run_body, pos0)

    def outer_cond(k):
      return k < nch

    def outer_body(k):
      cdrain(0)
      process(k, 0)

      @pl.when(k + 2 < nch)
      def _():
        cstart(k + 2, 0)

      @pl.when(k + 1 < nch)
      def _():
        cdrain(1)
        process(k + 1, 1)

      @pl.when(k + 3 < nch)
      def _():
        cstart(k + 3, 1)

      return k + 2

    lax.while_loop(outer_cond, outer_body, jnp.int32(0))

    tot = zf
    for s in range(SEG_PT):
      ts = jnp.sum(acc[pl.ds(s * LANES, LANES)])
      tot = tot + jnp.where(lane == s, ts, 0.0)
    part[...] = tot
    pltpu.sync_copy(part, out_hbm.at[wid])

  return seg_sum


_SEG_SUM = _make_sc_kernel()


@jax.jit
def kernel(inputs, labels):
  partials = _SEG_SUM(inputs, labels)
  return partials.reshape(-1)[:NSEG]
